# Initial kernel scaffold; baseline (speedup 1.0000x reference)
import jax
import jax.numpy as jnp
from jax.experimental import pallas as pl


def kernel(net_input, user_emb, rest_emb, W1, b1, W2, b2, W3, b3):
    raise NotImplementedError("write your pallas kernel here")


# speedup vs baseline: 1.4756x; 1.4756x over previous
import functools

import jax
import jax.numpy as jnp
from jax import lax
from jax.experimental import pallas as pl
from jax.experimental.pallas import tpu as pltpu
from jax.experimental.pallas import tpu_sc as plsc

EMB = 64
HIST = 50
NC, NS = 2, 16
NW = NC * NS
CH = 128
NQ = 4
BPW_DEFAULT = NQ * CH
LANES = 16


def _sc_gather_pool(idx_all, user_emb, rest_emb):
    B = idx_all.shape[0] * NQ * CH
    BPW = B // NW
    NT = HIST * NQ
    UROW = NT

    def body(idx_hbm, user_hbm, rest_hbm, out_u, out_s,
             idx_v, acc, ubuf, bufA, bufB, semU, semJ, semA, semB):
        wid = lax.axis_index("s") * NC + lax.axis_index("c")
        base = wid * BPW
        pltpu.sync_copy(idx_hbm.at[wid], idx_v)

        for q in range(NQ):
            pltpu.async_copy(user_hbm.at[idx_v.at[UROW + q]],
                             ubuf.at[pl.ds(q * CH, CH)], semU)
        for q in range(NQ):
            pltpu.async_copy(rest_hbm.at[idx_v.at[q]],
                             acc.at[pl.ds(q * CH, CH)], semJ)
        for q in range(NQ):
            pltpu.make_async_copy(user_hbm.at[idx_v.at[UROW + q]],
                                  ubuf.at[pl.ds(q * CH, CH)], semU).wait()
        u_out = pltpu.async_copy(ubuf, out_u.at[pl.ds(base, BPW)], semU)
        for q in range(NQ):
            pltpu.make_async_copy(rest_hbm.at[idx_v.at[q]],
                                  acc.at[pl.ds(q * CH, CH)], semJ).wait()

        bufs = (bufA, bufB)
        sems = (semA, semB)
        pltpu.async_copy(rest_hbm.at[idx_v.at[NQ]], bufA, semA)

        @pl.loop(NQ, NT, step=2)
        def _(g):
            for b in range(2):
                t = g + b
                nb = (b + 1) % 2

                @pl.when(t + 1 < NT)
                def _():
                    pltpu.async_copy(rest_hbm.at[idx_v.at[t + 1]],
                                     bufs[nb], sems[nb])

                pltpu.make_async_copy(rest_hbm.at[idx_v.at[t]],
                                      bufs[b], sems[b]).wait()
                qb = (t % NQ) * CH
                buf = bufs[b]

                @pl.loop(0, CH, unroll=4)
                def _(r):
                    for c in range(EMB // LANES):
                        sl = pl.ds(c * LANES, LANES)
                        plsc.addupdate(acc.at[qb + r, sl], buf[r, sl])

        pltpu.sync_copy(acc, out_s.at[pl.ds(base, BPW)])
        u_out.wait()

    f = pl.kernel(
        body,
        out_type=(jax.ShapeDtypeStruct((B, EMB), jnp.float32),
                  jax.ShapeDtypeStruct((B, EMB), jnp.float32)),
        mesh=plsc.VectorSubcoreMesh(core_axis_name="c", subcore_axis_name="s"),
        compiler_params=pltpu.CompilerParams(use_tc_tiling_on_sc=False),
        scratch_types=[
            pltpu.VMEM((NT + NQ, CH), jnp.int32),
            pltpu.VMEM((BPW, EMB), jnp.float32),
            pltpu.VMEM((BPW, EMB), jnp.float32),
            pltpu.VMEM((CH, EMB), jnp.float32),
            pltpu.VMEM((CH, EMB), jnp.float32),
            pltpu.SemaphoreType.DMA,
            pltpu.SemaphoreType.DMA,
            pltpu.SemaphoreType.DMA,
            pltpu.SemaphoreType.DMA,
        ],
    )
    return f(idx_all, user_emb, rest_emb)


def _mlp_body(u_ref, s_ref, w1u_ref, w1r_ref, b1_ref, w2_ref, b2_ref,
              w3_ref, b3_ref, o_ref):
    h1 = jnp.dot(u_ref[...], w1u_ref[...], preferred_element_type=jnp.float32)
    h1 += jnp.dot(s_ref[...], w1r_ref[...],
                  preferred_element_type=jnp.float32) * (1.0 / HIST)
    h1 = jnp.maximum(h1 + b1_ref[...], 0.0)
    h2 = jnp.dot(h1, w2_ref[...], preferred_element_type=jnp.float32)
    h2 = jnp.maximum(h2 + b2_ref[...], 0.0)
    y = jnp.dot(h2, w3_ref[...], preferred_element_type=jnp.float32)
    o_ref[...] = jax.nn.sigmoid(y + b3_ref[...])


def _tc_mlp(u, s, W1, b1, W2, b2, W3, b3):
    B = u.shape[0]
    H1, H2 = W1.shape[0], W2.shape[0]
    BLK = 2048
    grid = (B // BLK,)
    w1u = W1[:, :EMB].T
    w1r = W1[:, EMB:].T
    fixed = lambda i: (0, 0)
    return pl.pallas_call(
        _mlp_body,
        grid=grid,
        in_specs=[
            pl.BlockSpec((BLK, EMB), lambda i: (i, 0)),
            pl.BlockSpec((BLK, EMB), lambda i: (i, 0)),
            pl.BlockSpec((EMB, H1), fixed),
            pl.BlockSpec((EMB, H1), fixed),
            pl.BlockSpec((1, H1), fixed),
            pl.BlockSpec((H1, H2), fixed),
            pl.BlockSpec((1, H2), fixed),
            pl.BlockSpec((H2, 1), fixed),
            pl.BlockSpec((1, 1), fixed),
        ],
        out_specs=pl.BlockSpec((BLK, 1), lambda i: (i, 0)),
        out_shape=jax.ShapeDtypeStruct((B, 1), jnp.float32),
        compiler_params=pltpu.CompilerParams(
            dimension_semantics=("parallel",)),
    )(u, s, w1u, w1r, b1[None, :], W2.T, b2[None, :], W3.T, b3[None, :])


def kernel(net_input, user_emb, rest_emb, W1, b1, W2, b2, W3, b3):
    B = net_input.shape[0]
    rest = net_input[:, 1:]
    rest_t = (rest.reshape(NW, NQ, CH, HIST)
                  .transpose(0, 3, 1, 2)
                  .reshape(NW, HIST * NQ, CH))
    user_t = net_input[:, 0].reshape(NW, NQ, CH)
    idx_all = jnp.concatenate([rest_t, user_t], axis=1)
    u, s = _sc_gather_pool(idx_all, user_emb, rest_emb)
    return _tc_mlp(u, s, W1, b1, W2, b2, W3, b3)
